# baseline (speedup 1.0000x reference)
import functools

import jax
import jax.numpy as jnp
from jax import lax
from jax.experimental import pallas as pl
from jax.experimental.pallas import tpu as pltpu
from jax.experimental.pallas import tpu_sc as plsc

N = 10000
C = 128
E = 320000

NC = 2
NS = 16

NP = 10240
EP = 327680
CHUNK = 128
NCH = EP // (NC * NS * CHUNK)
HNCH = NCH // 2
ROWS_PT = NP // NS
RINIT = 400


def _sc_degree_body(dst_hbm, out_hbm, acc, dbuf, ones, zeros):
    c = lax.axis_index("c")
    s = lax.axis_index("s")
    for i in range(CHUNK // 16):
        ones[pl.ds(16 * i, 16)] = jnp.ones((16,), jnp.float32)
    for i in range(ROWS_PT // 16):
        zeros[pl.ds(16 * i, 16)] = jnp.zeros((16,), jnp.float32)
    pltpu.sync_copy(zeros, acc.at[pl.ds(s * ROWS_PT, ROWS_PT)])
    plsc.subcore_barrier()
    base = (c * NS + s) * NCH
    pltpu.sync_copy(dst_hbm.at[pl.ds(base, NCH)], dbuf)

    @pl.loop(0, NCH)
    def _(j):
        pltpu.sync_copy(ones, acc.at[dbuf.at[j]], add=True)

    plsc.subcore_barrier()
    pltpu.sync_copy(acc.at[pl.ds(s * ROWS_PT, ROWS_PT)],
                    out_hbm.at[c, 0, pl.ds(s * ROWS_PT, ROWS_PT)])


def _sc_aggregate_body(xs_hbm, src_hbm, dst_hbm, out_hbm,
                       acc, sbuf, dbuf, rows, gsems, ssems):
    c = lax.axis_index("c")
    s = lax.axis_index("s")
    pltpu.sync_copy(xs_hbm.at[pl.ds(s * ROWS_PT, RINIT)],
                    acc.at[pl.ds(s * ROWS_PT, RINIT)])

    @pl.when(s < NS - 1)
    def _():
        pltpu.sync_copy(xs_hbm.at[pl.ds(s * ROWS_PT + RINIT, ROWS_PT - RINIT)],
                        acc.at[pl.ds(s * ROWS_PT + RINIT, ROWS_PT - RINIT)])

    base = (c * NS + s) * NCH

    def start_gather(b, hj):
        pltpu.async_copy(xs_hbm.at[sbuf.at[hj]], rows.at[b], gsems.at[b])

    def wait_gather(b, hj):
        pltpu.make_async_copy(xs_hbm.at[sbuf.at[hj]], rows.at[b],
                              gsems.at[b]).wait()

    def start_scatter(b, hj):
        pltpu.async_copy(rows.at[b], acc.at[dbuf.at[hj]], ssems.at[b],
                         add=True)

    def wait_scatter(b, hj):
        pltpu.make_async_copy(rows.at[b], acc.at[dbuf.at[hj]],
                              ssems.at[b]).wait()

    for h in range(2):
        hbase = base + h * HNCH
        pltpu.sync_copy(src_hbm.at[pl.ds(hbase, HNCH)], sbuf)
        pltpu.sync_copy(dst_hbm.at[pl.ds(hbase, HNCH)], dbuf)
        if h == 0:
            for b in range(2):
                start_gather(b, b)
            plsc.subcore_barrier()
        else:
            for b in range(2):
                start_gather(b, b)

        @pl.loop(0, HNCH, step=2)
        def _(j):
            for b in range(2):
                wait_gather(b, j + b)
                start_scatter(b, j + b)
            for b in range(2):
                @pl.when(j + b + 2 < HNCH)
                def _():
                    wait_scatter(b, j + b)
                    start_gather(b, j + b + 2)

        for b in range(2):
            wait_scatter(b, HNCH - 2 + b)

    plsc.subcore_barrier()
    pltpu.sync_copy(acc.at[pl.ds(s * ROWS_PT, ROWS_PT)],
                    out_hbm.at[c, pl.ds(s * ROWS_PT, ROWS_PT)])


@functools.lru_cache(maxsize=None)
def _sc_kernels():
    mesh = plsc.VectorSubcoreMesh(
        core_axis_name="c", subcore_axis_name="s",
        num_cores=NC, num_subcores=NS)
    sc_degree = pl.kernel(
        _sc_degree_body,
        out_type=jax.ShapeDtypeStruct((NC, 1, NP), jnp.float32),
        mesh=mesh,
        scratch_types=[
            pltpu.VMEM_SHARED((NP,), jnp.float32),
            pltpu.VMEM((NCH, CHUNK), jnp.int32),
            pltpu.VMEM((CHUNK,), jnp.float32),
            pltpu.VMEM((ROWS_PT,), jnp.float32),
        ],
    )
    sc_aggregate = pl.kernel(
        _sc_aggregate_body,
        out_type=jax.ShapeDtypeStruct((NC, NP, C), jnp.float32),
        mesh=mesh,
        scratch_types=[
            pltpu.VMEM_SHARED((NP, C), jnp.float32),
            pltpu.VMEM((HNCH, CHUNK), jnp.int32),
            pltpu.VMEM((HNCH, CHUNK), jnp.int32),
            pltpu.VMEM((2, CHUNK, C), jnp.float32),
            pltpu.SemaphoreType.DMA((2,)),
            pltpu.SemaphoreType.DMA((2,)),
        ],
    )
    return sc_degree, sc_aggregate


def _tc_scale_body(deg_ref, x_ref, xs_ref, dis_ref):
    deg = deg_ref[:, 0:1] + deg_ref[:, 1:2] + 1.0
    dis = lax.rsqrt(deg)
    dis_ref[...] = dis
    xs_ref[...] = x_ref[...] * dis


def _tc_scale(deg_parts, x):
    r = 1000
    return pl.pallas_call(
        _tc_scale_body,
        grid=(N // r,),
        in_specs=[
            pl.BlockSpec((r, NC), lambda i: (i, 0)),
            pl.BlockSpec((r, C), lambda i: (i, 0)),
        ],
        out_specs=(
            pl.BlockSpec((r, C), lambda i: (i, 0)),
            pl.BlockSpec((r, 1), lambda i: (i, 0)),
        ),
        out_shape=(
            jax.ShapeDtypeStruct((N, C), jnp.float32),
            jax.ShapeDtypeStruct((N, 1), jnp.float32),
        ),
    )(deg_parts, x)


def _tc_layer_body(parts_ref, xs_ref, dis_ref, res_ref, w_ref, b_ref,
                   out_ref, xsn_ref=None):
    dis = dis_ref[...]
    u = (parts_ref[0] + parts_ref[1] - xs_ref[...]) * dis
    y = jnp.dot(u, w_ref[...], preferred_element_type=jnp.float32) + b_ref[...]
    a = y[:, :C]
    g = y[:, C:]
    o = a * jax.nn.sigmoid(g) + res_ref[...]
    out_ref[...] = o
    if xsn_ref is not None:
        xsn_ref[...] = o * dis


def _tc_layer(parts, xs, dis, res, w, b2d, want_next):
    r = 1000
    in_specs = [
        pl.BlockSpec((NC, r, C), lambda i: (0, i, 0)),
        pl.BlockSpec((r, C), lambda i: (i, 0)),
        pl.BlockSpec((r, 1), lambda i: (i, 0)),
        pl.BlockSpec((r, C), lambda i: (i, 0)),
        pl.BlockSpec((C, 2 * C), lambda i: (0, 0)),
        pl.BlockSpec((1, 2 * C), lambda i: (0, 0)),
    ]
    if want_next:
        body = _tc_layer_body
        out_specs = (pl.BlockSpec((r, C), lambda i: (i, 0)),
                     pl.BlockSpec((r, C), lambda i: (i, 0)))
        out_shape = (jax.ShapeDtypeStruct((N, C), jnp.float32),
                     jax.ShapeDtypeStruct((N, C), jnp.float32))
    else:
        def body(parts_ref, xs_ref, dis_ref, res_ref, w_ref, b_ref, out_ref):
            _tc_layer_body(parts_ref, xs_ref, dis_ref, res_ref, w_ref, b_ref,
                           out_ref)
        out_specs = pl.BlockSpec((r, C), lambda i: (i, 0))
        out_shape = jax.ShapeDtypeStruct((N, C), jnp.float32)
    return pl.pallas_call(
        body,
        grid=(N // r,),
        in_specs=in_specs,
        out_specs=out_specs,
        out_shape=out_shape,
    )(parts, xs, dis, res, w, b2d)


def kernel(x, edge_index, W0, b0, W1, b1):
    src = edge_index[0]
    dst = edge_index[1]
    npad = EP - E
    spread = jnp.arange(npad, dtype=jnp.int32) % (NP - N)
    srcp = jnp.concatenate([src, spread]).reshape(EP // CHUNK, CHUNK)
    dstp = jnp.concatenate([dst, N + spread]).reshape(EP // CHUNK, CHUNK)

    sc_degree, sc_aggregate = _sc_kernels()
    deg_parts = sc_degree(dstp)
    deg_parts = jnp.transpose(deg_parts[:, 0, :])
    xs1, dis = _tc_scale(deg_parts, x)
    parts1 = sc_aggregate(xs1, srcp, dstp)
    x1, xs2 = _tc_layer(parts1, xs1, dis, x, W0, b0.reshape(1, 2 * C), True)
    parts2 = sc_aggregate(xs2, srcp, dstp)
    return _tc_layer(parts2, xs2, dis, x1, W1, b1.reshape(1, 2 * C), False)


# speedup vs baseline: 1.2348x; 1.2348x over previous
import functools

import jax
import jax.numpy as jnp
from jax import lax
from jax.experimental import pallas as pl
from jax.experimental.pallas import tpu as pltpu
from jax.experimental.pallas import tpu_sc as plsc

N = 10000
C = 128
E = 320000

NC = 2
NS = 16

NP = 10240
EP = 327680
CHUNK = 128
NCH = EP // (NC * NS * CHUNK)
HNCH = NCH // 2
ROWS_PT = NP // NS
RINIT = 400


def _sc_degree_body(dst_hbm, out_hbm, acc, dbuf, ones, zeros):
    c = lax.axis_index("c")
    s = lax.axis_index("s")
    for i in range(CHUNK // 16):
        ones[pl.ds(16 * i, 16)] = jnp.ones((16,), jnp.float32)
    for i in range(ROWS_PT // 16):
        zeros[pl.ds(16 * i, 16)] = jnp.zeros((16,), jnp.float32)
    pltpu.sync_copy(zeros, acc.at[pl.ds(s * ROWS_PT, ROWS_PT)])
    plsc.subcore_barrier()
    base = (c * NS + s) * NCH
    pltpu.sync_copy(dst_hbm.at[pl.ds(base, NCH)], dbuf)

    @pl.loop(0, NCH)
    def _(j):
        pltpu.sync_copy(ones, acc.at[dbuf.at[j]], add=True)

    plsc.subcore_barrier()
    pltpu.sync_copy(acc.at[pl.ds(s * ROWS_PT, ROWS_PT)],
                    out_hbm.at[c, 0, pl.ds(s * ROWS_PT, ROWS_PT)])


def _sc_aggregate_body(xs_hbm, src_hbm, dst_hbm, out_hbm,
                       acc, sbuf, dbuf, rows, gsems):
    c = lax.axis_index("c")
    s = lax.axis_index("s")
    pltpu.sync_copy(xs_hbm.at[pl.ds(s * ROWS_PT, RINIT)],
                    acc.at[pl.ds(s * ROWS_PT, RINIT)])

    @pl.when(s < NS - 1)
    def _():
        pltpu.sync_copy(xs_hbm.at[pl.ds(s * ROWS_PT + RINIT, ROWS_PT - RINIT)],
                        acc.at[pl.ds(s * ROWS_PT + RINIT, ROWS_PT - RINIT)])

    base = (c * NS + s) * NCH
    rows0 = rows.at[0]
    rows1 = rows.at[1]
    sem0 = gsems.at[0]
    sem1 = gsems.at[1]

    for h in range(2):
        hbase = base + h * HNCH
        pltpu.sync_copy(src_hbm.at[pl.ds(hbase, HNCH)], sbuf)
        pltpu.sync_copy(dst_hbm.at[pl.ds(hbase, HNCH)], dbuf)
        pltpu.async_copy(xs_hbm.at[sbuf.at[0]], rows0, sem0)
        if h == 0:
            plsc.subcore_barrier()

        @pl.loop(0, HNCH, step=2)
        def _(j):
            pltpu.async_copy(xs_hbm.at[sbuf.at[j + 1]], rows1, sem1)
            pltpu.make_async_copy(xs_hbm.at[sbuf.at[j]], rows0, sem0).wait()
            pltpu.sync_copy(rows0, acc.at[dbuf.at[j]], add=True)

            @pl.when(j + 2 < HNCH)
            def _():
                pltpu.async_copy(xs_hbm.at[sbuf.at[j + 2]], rows0, sem0)

            pltpu.make_async_copy(xs_hbm.at[sbuf.at[j + 1]], rows1, sem1).wait()
            pltpu.sync_copy(rows1, acc.at[dbuf.at[j + 1]], add=True)

    plsc.subcore_barrier()
    pltpu.sync_copy(acc.at[pl.ds(s * ROWS_PT, ROWS_PT)],
                    out_hbm.at[c, pl.ds(s * ROWS_PT, ROWS_PT)])


@functools.lru_cache(maxsize=None)
def _sc_kernels():
    mesh = plsc.VectorSubcoreMesh(
        core_axis_name="c", subcore_axis_name="s",
        num_cores=NC, num_subcores=NS)
    sc_degree = pl.kernel(
        _sc_degree_body,
        out_type=jax.ShapeDtypeStruct((NC, 1, NP), jnp.float32),
        mesh=mesh,
        scratch_types=[
            pltpu.VMEM_SHARED((NP,), jnp.float32),
            pltpu.VMEM((NCH, CHUNK), jnp.int32),
            pltpu.VMEM((CHUNK,), jnp.float32),
            pltpu.VMEM((ROWS_PT,), jnp.float32),
        ],
    )
    sc_aggregate = pl.kernel(
        _sc_aggregate_body,
        out_type=jax.ShapeDtypeStruct((NC, NP, C), jnp.float32),
        mesh=mesh,
        scratch_types=[
            pltpu.VMEM_SHARED((NP, C), jnp.float32),
            pltpu.VMEM((HNCH, CHUNK), jnp.int32),
            pltpu.VMEM((HNCH, CHUNK), jnp.int32),
            pltpu.VMEM((2, CHUNK, C), jnp.float32),
            pltpu.SemaphoreType.DMA((2,)),
        ],
    )
    return sc_degree, sc_aggregate


def _tc_scale_body(deg_ref, x_ref, xs_ref, dis_ref):
    deg = deg_ref[:, 0:1] + deg_ref[:, 1:2] + 1.0
    dis = lax.rsqrt(deg)
    dis_ref[...] = dis
    xs_ref[...] = x_ref[...] * dis


def _tc_scale(deg_parts, x):
    r = 1000
    return pl.pallas_call(
        _tc_scale_body,
        grid=(N // r,),
        in_specs=[
            pl.BlockSpec((r, NC), lambda i: (i, 0)),
            pl.BlockSpec((r, C), lambda i: (i, 0)),
        ],
        out_specs=(
            pl.BlockSpec((r, C), lambda i: (i, 0)),
            pl.BlockSpec((r, 1), lambda i: (i, 0)),
        ),
        out_shape=(
            jax.ShapeDtypeStruct((N, C), jnp.float32),
            jax.ShapeDtypeStruct((N, 1), jnp.float32),
        ),
    )(deg_parts, x)


def _tc_layer_body(parts_ref, xs_ref, dis_ref, res_ref, w_ref, b_ref,
                   out_ref, xsn_ref=None):
    dis = dis_ref[...]
    u = (parts_ref[0] + parts_ref[1] - xs_ref[...]) * dis
    y = jnp.dot(u, w_ref[...], preferred_element_type=jnp.float32) + b_ref[...]
    a = y[:, :C]
    g = y[:, C:]
    o = a * jax.nn.sigmoid(g) + res_ref[...]
    out_ref[...] = o
    if xsn_ref is not None:
        xsn_ref[...] = o * dis


def _tc_layer(parts, xs, dis, res, w, b2d, want_next):
    r = 1000
    in_specs = [
        pl.BlockSpec((NC, r, C), lambda i: (0, i, 0)),
        pl.BlockSpec((r, C), lambda i: (i, 0)),
        pl.BlockSpec((r, 1), lambda i: (i, 0)),
        pl.BlockSpec((r, C), lambda i: (i, 0)),
        pl.BlockSpec((C, 2 * C), lambda i: (0, 0)),
        pl.BlockSpec((1, 2 * C), lambda i: (0, 0)),
    ]
    if want_next:
        body = _tc_layer_body
        out_specs = (pl.BlockSpec((r, C), lambda i: (i, 0)),
                     pl.BlockSpec((r, C), lambda i: (i, 0)))
        out_shape = (jax.ShapeDtypeStruct((N, C), jnp.float32),
                     jax.ShapeDtypeStruct((N, C), jnp.float32))
    else:
        def body(parts_ref, xs_ref, dis_ref, res_ref, w_ref, b_ref, out_ref):
            _tc_layer_body(parts_ref, xs_ref, dis_ref, res_ref, w_ref, b_ref,
                           out_ref)
        out_specs = pl.BlockSpec((r, C), lambda i: (i, 0))
        out_shape = jax.ShapeDtypeStruct((N, C), jnp.float32)
    return pl.pallas_call(
        body,
        grid=(N // r,),
        in_specs=in_specs,
        out_specs=out_specs,
        out_shape=out_shape,
    )(parts, xs, dis, res, w, b2d)


def kernel(x, edge_index, W0, b0, W1, b1):
    src = edge_index[0]
    dst = edge_index[1]
    npad = EP - E
    spread = jnp.arange(npad, dtype=jnp.int32) % (NP - N)
    srcp = jnp.concatenate([src, spread]).reshape(EP // CHUNK, CHUNK)
    dstp = jnp.concatenate([dst, N + spread]).reshape(EP // CHUNK, CHUNK)

    sc_degree, sc_aggregate = _sc_kernels()
    deg_parts = sc_degree(dstp)
    deg_parts = jnp.transpose(deg_parts[:, 0, :])
    xs1, dis = _tc_scale(deg_parts, x)
    parts1 = sc_aggregate(xs1, srcp, dstp)
    x1, xs2 = _tc_layer(parts1, xs1, dis, x, W0, b0.reshape(1, 2 * C), True)
    parts2 = sc_aggregate(xs2, srcp, dstp)
    return _tc_layer(parts2, xs2, dis, x1, W1, b1.reshape(1, 2 * C), False)


# speedup vs baseline: 1.2808x; 1.0373x over previous
import functools

import jax
import jax.numpy as jnp
from jax import lax
from jax.experimental import pallas as pl
from jax.experimental.pallas import tpu as pltpu
from jax.experimental.pallas import tpu_sc as plsc

N = 10000
C = 128
E = 320000

NC = 2
NS = 16
NW = NC * NS

CHUNK = 128
NCHE = E // CHUNK
TCH = NCHE // NW
XTRA = NCHE - TCH * NW
H0 = 40
H1 = TCH - H0
ROWS_PT = 640
RLAST = N - 15 * ROWS_PT
NPD = NS * ROWS_PT


def _sc_degree_body(dst_hbm, out_hbm, acc, dbuf, ones, zeros):
    c = lax.axis_index("c")
    s = lax.axis_index("s")
    tid = c * NS + s
    for i in range(CHUNK // 16):
        ones[pl.ds(16 * i, 16)] = jnp.ones((16,), jnp.float32)
    for i in range(ROWS_PT // 16):
        zeros[pl.ds(16 * i, 16)] = jnp.zeros((16,), jnp.float32)
    pltpu.sync_copy(zeros, acc.at[pl.ds(s * ROWS_PT, ROWS_PT)])
    plsc.subcore_barrier()
    pltpu.sync_copy(dst_hbm.at[pl.ds(tid * TCH, TCH)], dbuf)

    @pl.loop(0, TCH)
    def _(j):
        pltpu.sync_copy(ones, acc.at[dbuf.at[j, 0]], add=True)

    @pl.when(tid < XTRA)
    def _():
        pltpu.sync_copy(dst_hbm.at[pl.ds(NW * TCH + tid, 1)],
                        dbuf.at[pl.ds(0, 1)])
        pltpu.sync_copy(ones, acc.at[dbuf.at[0, 0]], add=True)

    plsc.subcore_barrier()
    pltpu.sync_copy(acc.at[pl.ds(s * ROWS_PT, ROWS_PT)],
                    out_hbm.at[c, 0, pl.ds(s * ROWS_PT, ROWS_PT)])


def _sc_aggregate_body(xs_hbm, src_hbm, dst_hbm, out_hbm,
                       acc, sbuf, dbuf, rows, gsems):
    c = lax.axis_index("c")
    s = lax.axis_index("s")
    tid = c * NS + s
    @pl.when(s < NS - 1)
    def _():
        pltpu.sync_copy(xs_hbm.at[pl.ds(s * ROWS_PT, ROWS_PT)],
                        acc.at[pl.ds(s * ROWS_PT, ROWS_PT)])

    @pl.when(s == NS - 1)
    def _():
        pltpu.sync_copy(xs_hbm.at[pl.ds(s * ROWS_PT, RLAST)],
                        acc.at[pl.ds(s * ROWS_PT, RLAST)])

    base = tid * TCH
    rows0 = rows.at[0]
    rows1 = rows.at[1]
    sem0 = gsems.at[0]
    sem1 = gsems.at[1]

    def run_phase(nch, first):
        pltpu.async_copy(xs_hbm.at[sbuf.at[0, 0]], rows0, sem0)
        if first:
            plsc.subcore_barrier()

        @pl.loop(0, nch, step=2)
        def _(j):
            pltpu.async_copy(xs_hbm.at[sbuf.at[j + 1, 0]], rows1, sem1)
            pltpu.make_async_copy(xs_hbm.at[sbuf.at[j, 0]], rows0,
                                  sem0).wait()
            pltpu.sync_copy(rows0, acc.at[dbuf.at[j, 0]], add=True)

            @pl.when(j + 2 < nch)
            def _():
                pltpu.async_copy(xs_hbm.at[sbuf.at[j + 2, 0]], rows0, sem0)

            pltpu.make_async_copy(xs_hbm.at[sbuf.at[j + 1, 0]], rows1,
                                  sem1).wait()
            pltpu.sync_copy(rows1, acc.at[dbuf.at[j + 1, 0]], add=True)

    pltpu.sync_copy(src_hbm.at[pl.ds(base, H0)], sbuf)
    pltpu.sync_copy(dst_hbm.at[pl.ds(base, H0)], dbuf)
    run_phase(H0, True)
    pltpu.sync_copy(src_hbm.at[pl.ds(base + H0, H1)], sbuf.at[pl.ds(0, H1)])
    pltpu.sync_copy(dst_hbm.at[pl.ds(base + H0, H1)], dbuf.at[pl.ds(0, H1)])
    run_phase(H1, False)

    @pl.when(tid < XTRA)
    def _():
        pltpu.sync_copy(src_hbm.at[pl.ds(NW * TCH + tid, 1)],
                        sbuf.at[pl.ds(0, 1)])
        pltpu.sync_copy(dst_hbm.at[pl.ds(NW * TCH + tid, 1)],
                        dbuf.at[pl.ds(0, 1)])
        pltpu.async_copy(xs_hbm.at[sbuf.at[0, 0]], rows0, sem0)
        pltpu.make_async_copy(xs_hbm.at[sbuf.at[0, 0]], rows0, sem0).wait()
        pltpu.sync_copy(rows0, acc.at[dbuf.at[0, 0]], add=True)

    plsc.subcore_barrier()

    @pl.when(s < NS - 1)
    def _():
        pltpu.sync_copy(acc.at[pl.ds(s * ROWS_PT, ROWS_PT)],
                        out_hbm.at[c, pl.ds(s * ROWS_PT, ROWS_PT)])

    @pl.when(s == NS - 1)
    def _():
        pltpu.sync_copy(acc.at[pl.ds(s * ROWS_PT, RLAST)],
                        out_hbm.at[c, pl.ds(s * ROWS_PT, RLAST)])


@functools.lru_cache(maxsize=None)
def _sc_kernels():
    mesh = plsc.VectorSubcoreMesh(
        core_axis_name="c", subcore_axis_name="s",
        num_cores=NC, num_subcores=NS)
    sc_degree = pl.kernel(
        _sc_degree_body,
        out_type=jax.ShapeDtypeStruct((NC, 1, NPD), jnp.float32),
        mesh=mesh,
        scratch_types=[
            pltpu.VMEM_SHARED((NPD,), jnp.float32),
            pltpu.VMEM((TCH, 1, CHUNK), jnp.int32),
            pltpu.VMEM((CHUNK,), jnp.float32),
            pltpu.VMEM((ROWS_PT,), jnp.float32),
        ],
    )
    sc_aggregate = pl.kernel(
        _sc_aggregate_body,
        out_type=jax.ShapeDtypeStruct((NC, N, C), jnp.float32),
        mesh=mesh,
        scratch_types=[
            pltpu.VMEM_SHARED((N, C), jnp.float32),
            pltpu.VMEM((H0, 1, CHUNK), jnp.int32),
            pltpu.VMEM((H0, 1, CHUNK), jnp.int32),
            pltpu.VMEM((2, CHUNK, C), jnp.float32),
            pltpu.SemaphoreType.DMA((2,)),
        ],
    )
    return sc_degree, sc_aggregate


def _tc_scale_body(deg_ref, x_ref, xs_ref, dis_ref):
    deg = deg_ref[:, 0:1] + deg_ref[:, 1:2] + 1.0
    dis = lax.rsqrt(deg)
    dis_ref[...] = dis
    xs_ref[...] = x_ref[...] * dis


def _tc_scale(deg_parts, x):
    r = 1000
    return pl.pallas_call(
        _tc_scale_body,
        grid=(N // r,),
        in_specs=[
            pl.BlockSpec((r, NC), lambda i: (i, 0)),
            pl.BlockSpec((r, C), lambda i: (i, 0)),
        ],
        out_specs=(
            pl.BlockSpec((r, C), lambda i: (i, 0)),
            pl.BlockSpec((r, 1), lambda i: (i, 0)),
        ),
        out_shape=(
            jax.ShapeDtypeStruct((N, C), jnp.float32),
            jax.ShapeDtypeStruct((N, 1), jnp.float32),
        ),
    )(deg_parts, x)


def _tc_layer_body(parts_ref, dis_ref, res_ref, w_ref, b_ref,
                   out_ref, xsn_ref=None):
    dis = dis_ref[...]
    res = res_ref[...]
    u = (parts_ref[0] + parts_ref[1]) * dis - res * (dis * dis)
    y = jnp.dot(u, w_ref[...], preferred_element_type=jnp.float32) + b_ref[...]
    a = y[:, :C]
    g = y[:, C:]
    o = a * jax.nn.sigmoid(g) + res
    out_ref[...] = o
    if xsn_ref is not None:
        xsn_ref[...] = o * dis


def _tc_layer(parts, dis, res, w, b2d, want_next):
    r = 1000
    in_specs = [
        pl.BlockSpec((NC, r, C), lambda i: (0, i, 0)),
        pl.BlockSpec((r, 1), lambda i: (i, 0)),
        pl.BlockSpec((r, C), lambda i: (i, 0)),
        pl.BlockSpec((C, 2 * C), lambda i: (0, 0)),
        pl.BlockSpec((1, 2 * C), lambda i: (0, 0)),
    ]
    if want_next:
        body = _tc_layer_body
        out_specs = (pl.BlockSpec((r, C), lambda i: (i, 0)),
                     pl.BlockSpec((r, C), lambda i: (i, 0)))
        out_shape = (jax.ShapeDtypeStruct((N, C), jnp.float32),
                     jax.ShapeDtypeStruct((N, C), jnp.float32))
    else:
        def body(parts_ref, dis_ref, res_ref, w_ref, b_ref, out_ref):
            _tc_layer_body(parts_ref, dis_ref, res_ref, w_ref, b_ref, out_ref)
        out_specs = pl.BlockSpec((r, C), lambda i: (i, 0))
        out_shape = jax.ShapeDtypeStruct((N, C), jnp.float32)
    return pl.pallas_call(
        body,
        grid=(N // r,),
        in_specs=in_specs,
        out_specs=out_specs,
        out_shape=out_shape,
    )(parts, dis, res, w, b2d)


def kernel(x, edge_index, W0, b0, W1, b1):
    srcp = edge_index[0].reshape(NCHE, 1, CHUNK)
    dstp = edge_index[1].reshape(NCHE, 1, CHUNK)

    sc_degree, sc_aggregate = _sc_kernels()
    deg_parts = sc_degree(dstp)
    deg_parts = jnp.transpose(deg_parts[:, 0, :])
    xs1, dis = _tc_scale(deg_parts, x)
    parts1 = sc_aggregate(xs1, srcp, dstp)
    x1, xs2 = _tc_layer(parts1, dis, x, W0, b0.reshape(1, 2 * C), True)
    parts2 = sc_aggregate(xs2, srcp, dstp)
    return _tc_layer(parts2, dis, x1, W1, b1.reshape(1, 2 * C), False)


# speedup vs baseline: 1.3878x; 1.0835x over previous
import functools

import jax
import jax.numpy as jnp
from jax import lax
from jax.experimental import pallas as pl
from jax.experimental.pallas import tpu as pltpu
from jax.experimental.pallas import tpu_sc as plsc

N = 10000
C = 128
E = 320000

NC = 2
NS = 16
NW = NC * NS

CHUNK = 128
NCHE = E // CHUNK
TCH = NCHE // NW
XTRA = NCHE - TCH * NW
PIPE = 12
NMAIN = (TCH // PIPE) * PIPE
TAIL = TCH - NMAIN
ISLOT = 6
ROWS_PT = 640
RLAST = N - 15 * ROWS_PT
NPD = NS * ROWS_PT


def _sc_degree_body(dst_hbm, out_hbm, acc, dbuf, ones, zeros):
    c = lax.axis_index("c")
    s = lax.axis_index("s")
    tid = c * NS + s
    for i in range(CHUNK // 16):
        ones[pl.ds(16 * i, 16)] = jnp.ones((16,), jnp.float32)
    for i in range(ROWS_PT // 16):
        zeros[pl.ds(16 * i, 16)] = jnp.zeros((16,), jnp.float32)
    pltpu.sync_copy(zeros, acc.at[pl.ds(s * ROWS_PT, ROWS_PT)])
    plsc.subcore_barrier()
    pltpu.sync_copy(dst_hbm.at[pl.ds(tid * TCH, TCH)], dbuf)

    @pl.loop(0, TCH)
    def _(j):
        pltpu.sync_copy(ones, acc.at[dbuf.at[j, 0]], add=True)

    @pl.when(tid < XTRA)
    def _():
        pltpu.sync_copy(dst_hbm.at[pl.ds(NW * TCH + tid, 1)],
                        dbuf.at[pl.ds(0, 1)])
        pltpu.sync_copy(ones, acc.at[dbuf.at[0, 0]], add=True)

    plsc.subcore_barrier()
    pltpu.sync_copy(acc.at[pl.ds(s * ROWS_PT, ROWS_PT)],
                    out_hbm.at[c, 0, pl.ds(s * ROWS_PT, ROWS_PT)])


def _sc_aggregate_body(xs_hbm, src_hbm, dst_hbm, out_hbm,
                       acc, sbuf, dbuf, rows, gsems, ssems, isems, idems):
    c = lax.axis_index("c")
    s = lax.axis_index("s")
    tid = c * NS + s
    @pl.when(s < NS - 1)
    def _():
        pltpu.sync_copy(xs_hbm.at[pl.ds(s * ROWS_PT, ROWS_PT)],
                        acc.at[pl.ds(s * ROWS_PT, ROWS_PT)])

    @pl.when(s == NS - 1)
    def _():
        pltpu.sync_copy(xs_hbm.at[pl.ds(s * ROWS_PT, RLAST)],
                        acc.at[pl.ds(s * ROWS_PT, RLAST)])

    base = tid * TCH

    def idx_load(j, q):
        pltpu.async_copy(src_hbm.at[pl.ds(base + j, 1)], sbuf.at[pl.ds(q, 1)],
                         isems.at[q])
        pltpu.async_copy(dst_hbm.at[pl.ds(base + j, 1)], dbuf.at[pl.ds(q, 1)],
                         idems.at[q])

    def idx_wait_s(j, q):
        pltpu.make_async_copy(src_hbm.at[pl.ds(base + j, 1)],
                              sbuf.at[pl.ds(q, 1)], isems.at[q]).wait()

    def idx_wait_d(j, q):
        pltpu.make_async_copy(dst_hbm.at[pl.ds(base + j, 1)],
                              dbuf.at[pl.ds(q, 1)], idems.at[q]).wait()

    def start_gather(jq, b):
        pltpu.async_copy(xs_hbm.at[sbuf.at[jq, 0]], rows.at[b], gsems.at[b])

    def wait_gather(jq, b):
        pltpu.make_async_copy(xs_hbm.at[sbuf.at[jq, 0]], rows.at[b],
                              gsems.at[b]).wait()

    def start_scatter(jq, b):
        pltpu.async_copy(rows.at[b], acc.at[dbuf.at[jq, 0]], ssems.at[b],
                         add=True)

    def wait_scatter(jq, b):
        pltpu.make_async_copy(rows.at[b], acc.at[dbuf.at[jq, 0]],
                              ssems.at[b]).wait()

    for q in range(3):
        idx_load(q, q)
    idx_wait_s(0, 0)
    start_gather(0, 0)
    idx_wait_s(1, 1)
    start_gather(1, 1)
    plsc.subcore_barrier()

    @pl.loop(0, NMAIN, step=PIPE)
    def _(jbase):
        for k in range(PIPE):
            j = jbase + k
            b = k % 3
            bg = (k + 2) % 3
            qg = (k + 2) % 4
            qn = (k + 3) % 4

            @pl.when(j + 2 < NMAIN)
            def _():
                idx_wait_s(j + 2, qg)

            @pl.when(jnp.logical_and(j >= 1, j + 2 < NMAIN))
            def _():
                wait_scatter(qn, bg)

            @pl.when(j + 2 < NMAIN)
            def _():
                start_gather(qg, bg)

            @pl.when(j + 3 < NMAIN)
            def _():
                idx_load(j + 3, qn)

            wait_gather(k % 4, b)
            idx_wait_d(j, k % 4)
            start_scatter(k % 4, b)

    for k in range(PIPE - 3, PIPE):
        wait_scatter(k % 4, k % 3)

    pltpu.sync_copy(src_hbm.at[pl.ds(base + NMAIN, TAIL)],
                    sbuf.at[pl.ds(0, TAIL)])
    pltpu.sync_copy(dst_hbm.at[pl.ds(base + NMAIN, TAIL)],
                    dbuf.at[pl.ds(0, TAIL)])
    start_gather(0, 0)
    for t in range(TAIL):
        if t + 1 < TAIL:
            start_gather(t + 1, (t + 1) % 2)
        wait_gather(t, t % 2)
        pltpu.sync_copy(rows.at[t % 2], acc.at[dbuf.at[t, 0]], add=True)

    @pl.when(tid < XTRA)
    def _():
        pltpu.sync_copy(src_hbm.at[pl.ds(NW * TCH + tid, 1)],
                        sbuf.at[pl.ds(0, 1)])
        pltpu.sync_copy(dst_hbm.at[pl.ds(NW * TCH + tid, 1)],
                        dbuf.at[pl.ds(0, 1)])
        start_gather(0, 0)
        wait_gather(0, 0)
        pltpu.sync_copy(rows.at[0], acc.at[dbuf.at[0, 0]], add=True)

    plsc.subcore_barrier()

    @pl.when(s < NS - 1)
    def _():
        pltpu.sync_copy(acc.at[pl.ds(s * ROWS_PT, ROWS_PT)],
                        out_hbm.at[c, pl.ds(s * ROWS_PT, ROWS_PT)])

    @pl.when(s == NS - 1)
    def _():
        pltpu.sync_copy(acc.at[pl.ds(s * ROWS_PT, RLAST)],
                        out_hbm.at[c, pl.ds(s * ROWS_PT, RLAST)])


@functools.lru_cache(maxsize=None)
def _sc_kernels():
    mesh = plsc.VectorSubcoreMesh(
        core_axis_name="c", subcore_axis_name="s",
        num_cores=NC, num_subcores=NS)
    sc_degree = pl.kernel(
        _sc_degree_body,
        out_type=jax.ShapeDtypeStruct((NC, 1, NPD), jnp.float32),
        mesh=mesh,
        scratch_types=[
            pltpu.VMEM_SHARED((NPD,), jnp.float32),
            pltpu.VMEM((TCH, 1, CHUNK), jnp.int32),
            pltpu.VMEM((CHUNK,), jnp.float32),
            pltpu.VMEM((ROWS_PT,), jnp.float32),
        ],
    )
    sc_aggregate = pl.kernel(
        _sc_aggregate_body,
        out_type=jax.ShapeDtypeStruct((NC, N, C), jnp.float32),
        mesh=mesh,
        scratch_types=[
            pltpu.VMEM_SHARED((N, C), jnp.float32),
            pltpu.VMEM((ISLOT, 1, CHUNK), jnp.int32),
            pltpu.VMEM((ISLOT, 1, CHUNK), jnp.int32),
            pltpu.VMEM((3, CHUNK, C), jnp.float32),
            pltpu.SemaphoreType.DMA((3,)),
            pltpu.SemaphoreType.DMA((3,)),
            pltpu.SemaphoreType.DMA((4,)),
            pltpu.SemaphoreType.DMA((4,)),
        ],
    )
    return sc_degree, sc_aggregate


def _tc_scale_body(deg_ref, x_ref, xs_ref, dis_ref):
    deg = deg_ref[:, 0:1] + deg_ref[:, 1:2] + 1.0
    dis = lax.rsqrt(deg)
    dis_ref[...] = dis
    xs_ref[...] = x_ref[...] * dis


def _tc_scale(deg_parts, x):
    r = 1000
    return pl.pallas_call(
        _tc_scale_body,
        grid=(N // r,),
        in_specs=[
            pl.BlockSpec((r, NC), lambda i: (i, 0)),
            pl.BlockSpec((r, C), lambda i: (i, 0)),
        ],
        out_specs=(
            pl.BlockSpec((r, C), lambda i: (i, 0)),
            pl.BlockSpec((r, 1), lambda i: (i, 0)),
        ),
        out_shape=(
            jax.ShapeDtypeStruct((N, C), jnp.float32),
            jax.ShapeDtypeStruct((N, 1), jnp.float32),
        ),
    )(deg_parts, x)


def _tc_layer_body(parts_ref, dis_ref, res_ref, w_ref, b_ref,
                   out_ref, xsn_ref=None):
    dis = dis_ref[...]
    res = res_ref[...]
    u = (parts_ref[0] + parts_ref[1]) * dis - res * (dis * dis)
    y = jnp.dot(u, w_ref[...], preferred_element_type=jnp.float32) + b_ref[...]
    a = y[:, :C]
    g = y[:, C:]
    o = a * jax.nn.sigmoid(g) + res
    out_ref[...] = o
    if xsn_ref is not None:
        xsn_ref[...] = o * dis


def _tc_layer(parts, dis, res, w, b2d, want_next):
    r = 1000
    in_specs = [
        pl.BlockSpec((NC, r, C), lambda i: (0, i, 0)),
        pl.BlockSpec((r, 1), lambda i: (i, 0)),
        pl.BlockSpec((r, C), lambda i: (i, 0)),
        pl.BlockSpec((C, 2 * C), lambda i: (0, 0)),
        pl.BlockSpec((1, 2 * C), lambda i: (0, 0)),
    ]
    if want_next:
        body = _tc_layer_body
        out_specs = (pl.BlockSpec((r, C), lambda i: (i, 0)),
                     pl.BlockSpec((r, C), lambda i: (i, 0)))
        out_shape = (jax.ShapeDtypeStruct((N, C), jnp.float32),
                     jax.ShapeDtypeStruct((N, C), jnp.float32))
    else:
        def body(parts_ref, dis_ref, res_ref, w_ref, b_ref, out_ref):
            _tc_layer_body(parts_ref, dis_ref, res_ref, w_ref, b_ref, out_ref)
        out_specs = pl.BlockSpec((r, C), lambda i: (i, 0))
        out_shape = jax.ShapeDtypeStruct((N, C), jnp.float32)
    return pl.pallas_call(
        body,
        grid=(N // r,),
        in_specs=in_specs,
        out_specs=out_specs,
        out_shape=out_shape,
    )(parts, dis, res, w, b2d)


def kernel(x, edge_index, W0, b0, W1, b1):
    srcp = edge_index[0].reshape(NCHE, 1, CHUNK)
    dstp = edge_index[1].reshape(NCHE, 1, CHUNK)

    sc_degree, sc_aggregate = _sc_kernels()
    deg_parts = sc_degree(dstp)
    deg_parts = jnp.transpose(deg_parts[:, 0, :])
    xs1, dis = _tc_scale(deg_parts, x)
    parts1 = sc_aggregate(xs1, srcp, dstp)
    x1, xs2 = _tc_layer(parts1, dis, x, W0, b0.reshape(1, 2 * C), True)
    parts2 = sc_aggregate(xs2, srcp, dstp)
    return _tc_layer(parts2, dis, x1, W1, b1.reshape(1, 2 * C), False)
